# baseline (device time: 6140 ns/iter reference)
import jax
import jax.numpy as jnp
from jax import lax
from jax.experimental import pallas as pl
from jax.experimental.pallas import tpu as pltpu


def kernel(x, dy, gamma):
    _, d = x.shape

    def body(x_hbm, dy_hbm, gamma_hbm, out_ref,
             x_v, dy_v, comm_ref, copy_sems, send_sem, recv_sem):
        my_x = lax.axis_index("x")
        my_y = lax.axis_index("y")
        peer = (my_x, 1 - my_y)

        barrier_sem = pltpu.get_barrier_semaphore()
        pl.semaphore_signal(
            barrier_sem, inc=1, device_id=peer,
            device_id_type=pl.DeviceIdType.MESH,
        )

        n_chunks = 4
        rows = x_v.shape[0] // n_chunks
        cps = []
        for c in range(n_chunks):
            sl = pl.ds(c * rows, rows)
            cp_x = pltpu.make_async_copy(
                x_hbm.at[sl], x_v.at[sl], copy_sems.at[2 * c])
            cp_dy = pltpu.make_async_copy(
                dy_hbm.at[sl], dy_v.at[sl], copy_sems.at[2 * c + 1])
            cp_x.start()
            cp_dy.start()
            cps.append((cp_x, cp_dy))

        inv_d = 1.0 / d
        dg = None
        db = None
        for c in range(n_chunks):
            cp_x, cp_dy = cps[c]
            sl = pl.ds(c * rows, rows)
            cp_x.wait()
            xv = x_v[sl, :]
            mu = jnp.sum(xv, axis=1, keepdims=True) * inv_d
            msq = jnp.sum(xv * xv, axis=1, keepdims=True) * inv_d
            rstd = lax.rsqrt(msq - mu * mu + 1e-5)
            cp_dy.wait()
            dyv = dy_v[sl, :]
            dg_c = jnp.sum((rstd * xv - mu * rstd) * dyv, axis=0)
            db_c = jnp.sum(dyv, axis=0)
            dg = dg_c if dg is None else dg + dg_c
            db = db_c if db is None else db + db_c
        comm_ref[0, 0, :] = dg
        comm_ref[0, 1, :] = db

        pl.semaphore_wait(barrier_sem, 1)

        rdma = pltpu.make_async_remote_copy(
            src_ref=comm_ref.at[0],
            dst_ref=comm_ref.at[1],
            send_sem=send_sem,
            recv_sem=recv_sem,
            device_id=peer,
            device_id_type=pl.DeviceIdType.MESH,
        )
        rdma.start()
        rdma.wait()

        out_ref[:, :] = comm_ref[0] + comm_ref[1]

    return pl.pallas_call(
        body,
        out_shape=jax.ShapeDtypeStruct((2, d), jnp.float32),
        in_specs=[pl.BlockSpec(memory_space=pltpu.MemorySpace.HBM)] * 3,
        out_specs=pl.BlockSpec(memory_space=pltpu.VMEM),
        scratch_shapes=[
            pltpu.VMEM((512, d), jnp.float32),
            pltpu.VMEM((512, d), jnp.float32),
            pltpu.VMEM((2, 2, d), jnp.float32),
            pltpu.SemaphoreType.DMA((8,)),
            pltpu.SemaphoreType.DMA,
            pltpu.SemaphoreType.DMA,
        ],
        compiler_params=pltpu.CompilerParams(collective_id=0),
    )(
        pltpu.with_memory_space_constraint(x, pltpu.MemorySpace.HBM),
        pltpu.with_memory_space_constraint(dy, pltpu.MemorySpace.HBM),
        pltpu.with_memory_space_constraint(gamma, pltpu.MemorySpace.HBM),
    )


# device time: 6115 ns/iter; 1.0041x vs baseline; 1.0041x over previous
import jax
import jax.numpy as jnp
from jax import lax
from jax.experimental import pallas as pl
from jax.experimental.pallas import tpu as pltpu


def kernel(x, dy, gamma):
    _, d = x.shape

    def body(x_hbm, dy_hbm, gamma_hbm, out_ref,
             x_v, dy_v, comm_ref, copy_sems, send_sem, recv_sem):
        my_x = lax.axis_index("x")
        my_y = lax.axis_index("y")
        peer = (my_x, 1 - my_y)

        barrier_sem = pltpu.get_barrier_semaphore()
        pl.semaphore_signal(
            barrier_sem, inc=1, device_id=peer,
            device_id_type=pl.DeviceIdType.MESH,
        )

        n_chunks = 2
        rows = x_v.shape[0] // n_chunks
        cps = []
        for c in range(n_chunks):
            sl = pl.ds(c * rows, rows)
            cp_x = pltpu.make_async_copy(
                x_hbm.at[sl], x_v.at[sl], copy_sems.at[2 * c])
            cp_dy = pltpu.make_async_copy(
                dy_hbm.at[sl], dy_v.at[sl], copy_sems.at[2 * c + 1])
            cp_x.start()
            cp_dy.start()
            cps.append((cp_x, cp_dy))

        inv_d = 1.0 / d
        dg = None
        db = None
        for c in range(n_chunks):
            cp_x, cp_dy = cps[c]
            sl = pl.ds(c * rows, rows)
            cp_x.wait()
            xv = x_v[sl, :]
            mu = jnp.sum(xv, axis=1, keepdims=True) * inv_d
            msq = jnp.sum(xv * xv, axis=1, keepdims=True) * inv_d
            rstd = lax.rsqrt(msq - mu * mu + 1e-5)
            cp_dy.wait()
            dyv = dy_v[sl, :]
            dg_c = jnp.sum((rstd * xv - mu * rstd) * dyv, axis=0)
            db_c = jnp.sum(dyv, axis=0)
            dg = dg_c if dg is None else dg + dg_c
            db = db_c if db is None else db + db_c
        comm_ref[0, 0, :] = dg
        comm_ref[0, 1, :] = db

        pl.semaphore_wait(barrier_sem, 1)

        rdma = pltpu.make_async_remote_copy(
            src_ref=comm_ref.at[0],
            dst_ref=comm_ref.at[1],
            send_sem=send_sem,
            recv_sem=recv_sem,
            device_id=peer,
            device_id_type=pl.DeviceIdType.MESH,
        )
        rdma.start()
        rdma.wait()

        out_ref[:, :] = comm_ref[0] + comm_ref[1]

    return pl.pallas_call(
        body,
        out_shape=jax.ShapeDtypeStruct((2, d), jnp.float32),
        in_specs=[pl.BlockSpec(memory_space=pltpu.MemorySpace.HBM)] * 3,
        out_specs=pl.BlockSpec(memory_space=pltpu.VMEM),
        scratch_shapes=[
            pltpu.VMEM((512, d), jnp.float32),
            pltpu.VMEM((512, d), jnp.float32),
            pltpu.VMEM((2, 2, d), jnp.float32),
            pltpu.SemaphoreType.DMA((4,)),
            pltpu.SemaphoreType.DMA,
            pltpu.SemaphoreType.DMA,
        ],
        compiler_params=pltpu.CompilerParams(collective_id=0),
    )(
        pltpu.with_memory_space_constraint(x, pltpu.MemorySpace.HBM),
        pltpu.with_memory_space_constraint(dy, pltpu.MemorySpace.HBM),
        pltpu.with_memory_space_constraint(gamma, pltpu.MemorySpace.HBM),
    )
